# Initial kernel scaffold; baseline (speedup 1.0000x reference)
import jax
import jax.numpy as jnp
from jax.experimental import pallas as pl


def kernel(x, emb1_weight, emb2_weight):
    raise NotImplementedError("write your pallas kernel here")


# speedup vs baseline: 3.1861x; 3.1861x over previous
import jax
import jax.numpy as jnp
from jax.experimental import pallas as pl
from jax.experimental.pallas import tpu as pltpu
from jax.experimental.pallas import tpu_sc as plsc

_B = 4096
_L = 200
_D = 192
_N = _B * _L
_WINDOW = 128
_NUM_SUBCORES = 32
_WINDOWS_PER_SUBCORE = _N // (_WINDOW * _NUM_SUBCORES)


def _gather_body(tbl_hbm, idx_hbm, out_hbm, idx_ref, row_ref, sem_i, sem_g, sem_o):
    core = jax.lax.axis_index("core")
    sub = jax.lax.axis_index("subcore")
    sid = core * 16 + sub

    @pl.loop(0, _WINDOWS_PER_SUBCORE)
    def _(w):
        base = (sid * _WINDOWS_PER_SUBCORE + w) * _WINDOW
        pltpu.async_copy(idx_hbm.at[pl.ds(base, _WINDOW)], idx_ref, sem_i).wait()
        pltpu.async_copy(tbl_hbm.at[idx_ref], row_ref, sem_g).wait()
        pltpu.async_copy(row_ref, out_hbm.at[pl.ds(base, _WINDOW)], sem_o).wait()


def kernel(x, emb1_weight, emb2_weight):
    table = jnp.concatenate((emb1_weight, emb2_weight), axis=1)
    idx = x.reshape(_N).astype(jnp.int32)

    gather = pl.kernel(
        _gather_body,
        out_type=jax.ShapeDtypeStruct((_N, _D), jnp.float32),
        mesh=plsc.VectorSubcoreMesh(
            core_axis_name="core", subcore_axis_name="subcore"
        ),
        scratch_types=[
            pltpu.VMEM((_WINDOW,), jnp.int32),
            pltpu.VMEM((_WINDOW, _D), jnp.float32),
            pltpu.SemaphoreType.DMA,
            pltpu.SemaphoreType.DMA,
            pltpu.SemaphoreType.DMA,
        ],
        compiler_params=pltpu.CompilerParams(use_tc_tiling_on_sc=False),
    )
    out = gather(table, idx)
    return out.reshape(_B, 1, _L, _D)


# speedup vs baseline: 3.4256x; 1.0752x over previous
import jax
import jax.numpy as jnp
from jax.experimental import pallas as pl
from jax.experimental.pallas import tpu as pltpu
from jax.experimental.pallas import tpu_sc as plsc

_B = 4096
_L = 200
_D = 192
_N = _B * _L
_WINDOW = 128


def _gather_body(tbl_hbm, idx_hbm, out_hbm):
    def body(idx_vmem, out_vmem):
        pltpu.sync_copy(tbl_hbm.at[idx_vmem.at[0]], out_vmem)

    pltpu.emit_pipeline(
        body,
        grid=(_N // _WINDOW,),
        in_specs=[pl.BlockSpec((1, _WINDOW), lambda i: (0, i))],
        out_specs=[pl.BlockSpec((_WINDOW, _D), lambda i: (i, 0))],
        core_axis_name=("core", "subcore"),
        dimension_semantics=(pltpu.PARALLEL,),
    )(idx_hbm, out_hbm)


def kernel(x, emb1_weight, emb2_weight):
    table = jnp.concatenate((emb1_weight, emb2_weight), axis=1)
    idx = x.reshape(1, _N).astype(jnp.int32)

    gather = pl.kernel(
        _gather_body,
        out_type=jax.ShapeDtypeStruct((_N, _D), jnp.float32),
        mesh=plsc.VectorSubcoreMesh(
            core_axis_name="core", subcore_axis_name="subcore"
        ),
        compiler_params=pltpu.CompilerParams(use_tc_tiling_on_sc=False),
    )
    out = gather(table, idx)
    return out.reshape(_B, 1, _L, _D)


# speedup vs baseline: 3.4333x; 1.0022x over previous
import jax
import jax.numpy as jnp
from jax.experimental import pallas as pl
from jax.experimental.pallas import tpu as pltpu
from jax.experimental.pallas import tpu_sc as plsc

_B = 4096
_L = 200
_D = 192
_N = _B * _L
_W = 128
_NBUF = 4
_NSUB = 32
_WPS = _N // (_W * _NSUB)
_IPS = _N // _NSUB
_GROUPS = _WPS // _NBUF


def _gather_body(tbl_hbm, idx_hbm, out_hbm, idx_ref, r0, r1, r2, r3, si, gs, ws):
    rows = [r0, r1, r2, r3]
    core = jax.lax.axis_index("core")
    sub = jax.lax.axis_index("subcore")
    sid = core * 16 + sub
    wbase = sid * _WPS

    pltpu.async_copy(idx_hbm.at[pl.ds(sid * _IPS, _IPS)], idx_ref, si).wait()

    def start_gather(w, b):
        pltpu.async_copy(
            tbl_hbm.at[idx_ref.at[pl.ds(w * _W, _W)]], rows[b], gs.at[b]
        )

    def wait_gather(b):
        pltpu.make_async_copy(
            tbl_hbm.at[idx_ref.at[pl.ds(0, _W)]], rows[b], gs.at[b]
        ).wait()

    def start_write(w, b):
        pltpu.async_copy(
            rows[b], out_hbm.at[pl.ds((wbase + w) * _W, _W)], ws.at[b]
        )

    def wait_write(b):
        pltpu.make_async_copy(
            rows[b], out_hbm.at[pl.ds(wbase * _W, _W)], ws.at[b]
        ).wait()

    for b in range(_NBUF):
        start_gather(b, b)

    @pl.loop(1, _GROUPS)
    def _(g):
        for b in range(_NBUF):
            wait_gather(b)
            start_write((g - 1) * _NBUF + b, b)
        for b in range(_NBUF):
            wait_write(b)
            start_gather(g * _NBUF + b, b)

    for b in range(_NBUF):
        wait_gather(b)
        start_write((_GROUPS - 1) * _NBUF + b, b)
    for b in range(_NBUF):
        wait_write(b)


def kernel(x, emb1_weight, emb2_weight):
    table = jnp.concatenate((emb1_weight, emb2_weight), axis=1)
    idx = x.reshape(_N).astype(jnp.int32)

    gather = pl.kernel(
        _gather_body,
        out_type=jax.ShapeDtypeStruct((_N, _D), jnp.float32),
        mesh=plsc.VectorSubcoreMesh(
            core_axis_name="core", subcore_axis_name="subcore"
        ),
        scratch_types=[
            pltpu.VMEM((_IPS,), jnp.int32),
            pltpu.VMEM((_W, _D), jnp.float32),
            pltpu.VMEM((_W, _D), jnp.float32),
            pltpu.VMEM((_W, _D), jnp.float32),
            pltpu.VMEM((_W, _D), jnp.float32),
            pltpu.SemaphoreType.DMA,
            pltpu.SemaphoreType.DMA((_NBUF,)),
            pltpu.SemaphoreType.DMA((_NBUF,)),
        ],
        compiler_params=pltpu.CompilerParams(use_tc_tiling_on_sc=False),
    )
    out = gather(table, idx)
    return out.reshape(_B, 1, _L, _D)


# speedup vs baseline: 4.9129x; 1.4310x over previous
import jax
import jax.numpy as jnp
from jax.experimental import pallas as pl
from jax.experimental.pallas import tpu as pltpu
from jax.experimental.pallas import tpu_sc as plsc

_B = 4096
_L = 200
_D = 192
_N = _B * _L
_W = 128
_NBUF = 2
_NSUB = 32
_WPS = _N // (_W * _NSUB)
_IPS = _N // _NSUB
_GROUPS = _WPS // _NBUF


def _gather_body(
    tlo_hbm, thi_hbm, idx_hbm, out_hbm,
    idx_ref, lo0, lo1, hi0, hi1, h640, h641,
    si, gs, ws,
):
    los = [lo0, lo1]
    his = [hi0, hi1]
    h64s = [h640, h641]
    core = jax.lax.axis_index("core")
    sub = jax.lax.axis_index("subcore")
    sid = core * 16 + sub
    wbase = sid * _WPS

    pltpu.async_copy(idx_hbm.at[pl.ds(sid * _IPS, _IPS)], idx_ref, si).wait()

    def start_gathers(w, b):
        iv = idx_ref.at[pl.ds(w * _W, _W)]
        pltpu.async_copy(tlo_hbm.at[iv], los[b], gs.at[b])
        pltpu.async_copy(thi_hbm.at[iv], his[b], gs.at[b])

    def wait_gathers(b):
        iv = idx_ref.at[pl.ds(0, _W)]
        pltpu.make_async_copy(tlo_hbm.at[iv], los[b], gs.at[b]).wait()
        pltpu.make_async_copy(thi_hbm.at[iv], his[b], gs.at[b]).wait()

    def repack(b):
        @pl.loop(0, _W)
        def _(r):
            for j in range(4):
                src = (pl.ds(r, 1), pl.ds(64 + j * 16, 16))
                dst = (pl.ds(r, 1), pl.ds(j * 16, 16))
                h64s[b].at[dst][...] = his[b].at[src][...]

    def start_writes(w, b):
        rows = pl.ds((wbase + w) * _W, _W)
        pltpu.async_copy(los[b], out_hbm.at[rows, pl.ds(0, 128)], ws.at[b])
        pltpu.async_copy(h64s[b], out_hbm.at[rows, pl.ds(128, 64)], ws.at[b])

    def wait_writes(b):
        rows = pl.ds(wbase * _W, _W)
        pltpu.make_async_copy(los[b], out_hbm.at[rows, pl.ds(0, 128)], ws.at[b]).wait()
        pltpu.make_async_copy(h64s[b], out_hbm.at[rows, pl.ds(128, 64)], ws.at[b]).wait()

    for b in range(_NBUF):
        start_gathers(b, b)

    @pl.loop(1, _GROUPS)
    def _(g):
        for b in range(_NBUF):
            wait_gathers(b)
            repack(b)
            start_writes((g - 1) * _NBUF + b, b)
        for b in range(_NBUF):
            wait_writes(b)
            start_gathers(g * _NBUF + b, b)

    for b in range(_NBUF):
        wait_gathers(b)
        repack(b)
        start_writes((_GROUPS - 1) * _NBUF + b, b)
    for b in range(_NBUF):
        wait_writes(b)


def kernel(x, emb1_weight, emb2_weight):
    tlo = jnp.concatenate((emb1_weight, emb2_weight[:, :64]), axis=1)
    idx = x.reshape(_N).astype(jnp.int32)

    gather = pl.kernel(
        _gather_body,
        out_type=jax.ShapeDtypeStruct((_N, _D), jnp.float32),
        mesh=plsc.VectorSubcoreMesh(
            core_axis_name="core", subcore_axis_name="subcore"
        ),
        scratch_types=[
            pltpu.VMEM((_IPS,), jnp.int32),
            pltpu.VMEM((_W, 128), jnp.float32),
            pltpu.VMEM((_W, 128), jnp.float32),
            pltpu.VMEM((_W, 128), jnp.float32),
            pltpu.VMEM((_W, 128), jnp.float32),
            pltpu.VMEM((_W, 64), jnp.float32),
            pltpu.VMEM((_W, 64), jnp.float32),
            pltpu.SemaphoreType.DMA,
            pltpu.SemaphoreType.DMA((_NBUF,)),
            pltpu.SemaphoreType.DMA((_NBUF,)),
        ],
    )
    out = gather(tlo, emb2_weight, idx)
    return out.reshape(_B, 1, _L, _D)
